# baseline (speedup 1.0000x reference)
import jax
import jax.numpy as jnp
from jax.experimental import pallas as pl

_Q = 0.7
_scale_cache = {}


def _get_scale(n, dtype):
    key = (n, dtype)
    if key not in _scale_cache:
        mask = jax.random.bernoulli(jax.random.key(42), _Q, (n,))
        _scale_cache[key] = jnp.where(mask, 0.0, 1.0).astype(dtype)[:, None]
    return _scale_cache[key]


def _mask_body(x_ref, s_ref, o_ref):
    o_ref[...] = x_ref[...] * s_ref[...]


def kernel(x):
    n, d = x.shape
    scale = _get_scale(n, x.dtype)
    br = 4000
    return pl.pallas_call(
        _mask_body,
        grid=(n // br,),
        in_specs=[
            pl.BlockSpec((br, d), lambda i: (i, 0)),
            pl.BlockSpec((br, 1), lambda i: (i, 0)),
        ],
        out_specs=pl.BlockSpec((br, d), lambda i: (i, 0)),
        out_shape=jax.ShapeDtypeStruct((n, d), x.dtype),
    )(x, scale)


# speedup vs baseline: 1.0001x; 1.0001x over previous
import jax
import jax.numpy as jnp
from jax.experimental import pallas as pl
from jax.experimental.pallas import tpu as pltpu

_Q = 0.7
_scale_cache = {}


def _get_scale(n, dtype):
    key = (n, dtype)
    if key not in _scale_cache:
        mask = jax.random.bernoulli(jax.random.key(42), _Q, (n,))
        _scale_cache[key] = jnp.where(mask, 0.0, 1.0).astype(dtype)[:, None]
    return _scale_cache[key]


def _mask_body(x_ref, s_ref, o_ref):
    o_ref[...] = x_ref[...] * s_ref[...]


def kernel(x):
    n, d = x.shape
    scale = _get_scale(n, x.dtype)
    br = 4000
    return pl.pallas_call(
        _mask_body,
        grid=(n // br,),
        in_specs=[
            pl.BlockSpec((br, d), lambda i: (i, 0)),
            pl.BlockSpec((br, 1), lambda i: (i, 0)),
        ],
        out_specs=pl.BlockSpec((br, d), lambda i: (i, 0)),
        out_shape=jax.ShapeDtypeStruct((n, d), x.dtype),
        compiler_params=pltpu.CompilerParams(
            dimension_semantics=("parallel",),
        ),
    )(x, scale)
